# baseline (speedup 1.0000x reference)
import dataclasses
import functools

import jax
import jax.numpy as jnp
from jax import lax
from jax.experimental import pallas as pl
from jax.experimental.pallas import tpu as pltpu
from jax.experimental.pallas import tpu_sc as plsc

NC = 2
NS = 16
LANES = 16
NW = NC * NS
GK = 128
EB = 2000
SCAN_UNROLL = 5


def _sc_aggregate(x, dst, src, ew, n_pad, d):
    e = dst.shape[0]
    assert e % EB == 0 and EB % (LANES * SCAN_UNROLL) == 0
    nb = e // EB
    assert nb % 2 == 0
    rb = n_pad // NW
    assert rb % 8 == 0
    mc = ((EB + 2 * GK + GK - 1) // GK) * GK
    dch = d // LANES

    mesh = plsc.VectorSubcoreMesh(core_axis_name="c", subcore_axis_name="s")
    cp = pltpu.CompilerParams()
    if "needs_layout_passes" in pltpu.CompilerParams.__dataclass_fields__:
        cp = dataclasses.replace(cp, needs_layout_passes=False)

    @functools.partial(
        pl.kernel,
        mesh=mesh,
        compiler_params=cp,
        out_type=jax.ShapeDtypeStruct((n_pad, d), jnp.float32),
        scratch_types=[
            pltpu.VMEM((rb, d), jnp.float32),
            pltpu.VMEM((EB,), jnp.int32),
            pltpu.VMEM((EB,), jnp.int32),
            pltpu.VMEM((EB,), jnp.int32),
            pltpu.VMEM((EB,), jnp.int32),
            pltpu.VMEM((EB,), jnp.float32),
            pltpu.VMEM((EB,), jnp.float32),
            pltpu.VMEM((mc,), jnp.int32),
            pltpu.VMEM((mc,), jnp.int32),
            pltpu.VMEM((mc // GK, GK), jnp.int32),
            pltpu.VMEM((mc // GK, GK), jnp.int32),
            pltpu.VMEM((mc,), jnp.float32),
            pltpu.VMEM((mc,), jnp.float32),
            pltpu.VMEM((GK, d), jnp.float32),
            pltpu.VMEM((GK, d), jnp.float32),
            pltpu.VMEM((LANES,), jnp.int32),
            pltpu.VMEM((LANES,), jnp.int32),
            pltpu.SemaphoreType.DMA,
            pltpu.SemaphoreType.DMA,
            pltpu.SemaphoreType.DMA,
            pltpu.SemaphoreType.DMA,
        ],
    )
    def agg_kernel(x_hbm, dst_hbm, src_hbm, ew_hbm, out_hbm,
                   acc_v, sdst_a, sdst_b, ssrc_a, ssrc_b, sw_a, sw_b,
                   mdst_a, mdst_b, msrc_a, msrc_b, mw_a, mw_b,
                   rows_a, rows_b, cnt_a, cnt_b,
                   sem_sa, sem_sb, sem_ga, sem_gb):
        sdst = (sdst_a, sdst_b)
        ssrc = (ssrc_a, ssrc_b)
        sw = (sw_a, sw_b)
        mdst = (mdst_a, mdst_b)
        msrc = (msrc_a, msrc_b)
        mw = (mw_a, mw_b)
        rows = (rows_a, rows_b)
        cnts = (cnt_a, cnt_b)
        c = lax.axis_index("c")
        s = lax.axis_index("s")
        wid = c * NS + s
        lo = wid * rb
        iota = lax.iota(jnp.int32, LANES)

        @pl.loop(0, rb)
        def _(r):
            for ch in range(dch):
                acc_v[r, pl.ds(ch * LANES, LANES)] = jnp.zeros((LANES,), jnp.float32)

        def fire_staging(b, p, sem):
            off = b * EB
            pltpu.async_copy(dst_hbm.at[pl.ds(off, EB)], sdst[p], sem)
            pltpu.async_copy(src_hbm.at[pl.ds(off, EB)], ssrc[p], sem)
            pltpu.async_copy(ew_hbm.at[pl.ds(off, EB)], sw[p], sem)

        def wait_staging(p, sem):
            pltpu.make_async_copy(dst_hbm.at[pl.ds(0, EB)], sdst[p], sem).wait()
            pltpu.make_async_copy(src_hbm.at[pl.ds(0, EB)], ssrc[p], sem).wait()
            pltpu.make_async_copy(ew_hbm.at[pl.ds(0, EB)], sw[p], sem).wait()

        def fire_gather(p, sem):
            pltpu.async_copy(x_hbm.at[msrc[p].at[0]], rows[p], sem)

        def wait_gather(p, sem):
            pltpu.make_async_copy(x_hbm.at[msrc[p].at[0]], rows[p], sem).wait()

        def scan_block(p):
            def chunk(k, ptr):
                for u in range(SCAN_UNROLL):
                    off = (k * SCAN_UNROLL + u) * LANES
                    t = sdst[p][pl.ds(off, LANES)] - lo
                    mask = (t >= 0) & (t < rb)
                    pos = ptr + plsc.cumsum(mask.astype(jnp.int32))
                    plsc.store_scatter(mdst[p], [pos], t, mask=mask)
                    plsc.store_scatter(msrc[p],
                                       [lax.shift_right_logical(pos, 7),
                                        lax.bitwise_and(pos, GK - 1)],
                                       ssrc[p][pl.ds(off, LANES)], mask=mask)
                    plsc.store_scatter(mw[p], [pos],
                                       sw[p][pl.ds(off, LANES)], mask=mask)
                    ptr = ptr + plsc.all_reduce_population_count(mask)
                return ptr

            ptr = lax.fori_loop(0, EB // (LANES * SCAN_UNROLL), chunk,
                                jnp.full((LANES,), -1, jnp.int32))
            cnts[p][pl.ds(0, LANES)] = ptr + 1
            m = (ptr + 1)[0]
            for k in range(GK // LANES):
                posz = m + k * LANES + iota
                plsc.store_scatter(msrc[p],
                                   [lax.shift_right_logical(posz, 7),
                                    lax.bitwise_and(posz, GK - 1)],
                                   jnp.zeros((LANES,), jnp.int32))
            return m

        def edge_chunk(p, base, cnt):
            def edge(t2, _):
                g = base + t2
                dloc = mdst[p][pl.ds(g, LANES)][0]
                wsp = mw[p][pl.ds(g, LANES)][0]
                for ch in range(dch):
                    sl = pl.ds(ch * LANES, LANES)
                    plsc.addupdate(acc_v.at[dloc, sl], rows[p][t2, sl] * wsp)
                return 0

            lax.fori_loop(0, cnt, edge, 0)

        def process_block(p):
            m = cnts[p][pl.ds(0, LANES)][0]
            edge_chunk(p, 0, jnp.minimum(m, GK))
            for j in range(1, EB // GK):
                @pl.when(m > j * GK)
                def _(j=j):
                    pltpu.sync_copy(x_hbm.at[msrc[p].at[j]], rows[p])
                    edge_chunk(p, j * GK, jnp.minimum(m - j * GK, GK))

        sems = (sem_sa, sem_sb)
        gsems = (sem_ga, sem_gb)
        fire_staging(0, 0, sem_sa)

        @pl.loop(0, nb // 2)
        def _(q):
            for step in range(2):
                b = q * 2 + step
                p, po = step, 1 - step
                wait_staging(p, sems[p])
                if step == 0:
                    fire_staging(b + 1, po, sems[po])
                else:
                    @pl.when(q < nb // 2 - 1)
                    def _():
                        fire_staging(b + 1, po, sems[po])
                scan_block(p)
                fire_gather(p, gsems[p])
                if step == 0:
                    @pl.when(q > 0)
                    def _():
                        wait_gather(po, gsems[po])
                        process_block(po)
                else:
                    wait_gather(po, gsems[po])
                    process_block(po)

        wait_gather(1, sem_gb)
        process_block(1)

        pltpu.sync_copy(acc_v, out_hbm.at[pl.ds(lo, rb)])

    return agg_kernel(x, dst, src, ew)


def _tc_finish(agg, W, bias, n_out, blk=1000):
    d = agg.shape[1]
    d_out = W.shape[1]

    def body(p_ref, w_ref, b_ref, o_ref):
        o_ref[...] = jnp.dot(p_ref[...], w_ref[...],
                             preferred_element_type=jnp.float32) + b_ref[...]

    return pl.pallas_call(
        body,
        grid=(n_out // blk,),
        in_specs=[
            pl.BlockSpec((blk, d), lambda i: (i, 0)),
            pl.BlockSpec((d, d_out), lambda i: (0, 0)),
            pl.BlockSpec((1, d_out), lambda i: (0, 0)),
        ],
        out_specs=pl.BlockSpec((blk, d_out), lambda i: (i, 0)),
        out_shape=jax.ShapeDtypeStruct((n_out, d_out), jnp.float32),
    )(agg, W, bias.reshape(1, d_out))


def kernel(x, edge_index, edge_weight, W, bias):
    n, d = x.shape
    n_pad = ((n + NW * 8 - 1) // (NW * 8)) * NW * 8
    agg = _sc_aggregate(x, edge_index[0], edge_index[1], edge_weight, n_pad, d)
    return _tc_finish(agg, W, bias, n)


# speedup vs baseline: 25.5320x; 25.5320x over previous
import dataclasses
import functools

import jax
import jax.numpy as jnp
from jax import lax
from jax.experimental import pallas as pl
from jax.experimental.pallas import tpu as pltpu
from jax.experimental.pallas import tpu_sc as plsc

NC = 2
NS = 16
LANES = 16
GK = 128


def _sc_aggregate(x, src, dst, ew, n_pad, d, n_chunks):
    rows_per_tile = n_pad // NS
    assert rows_per_tile % 8 == 0
    mesh = plsc.VectorSubcoreMesh(core_axis_name="c", subcore_axis_name="s")
    cp = pltpu.CompilerParams()
    if "needs_layout_passes" in pltpu.CompilerParams.__dataclass_fields__:
        cp = dataclasses.replace(cp, needs_layout_passes=False)

    @functools.partial(
        pl.kernel,
        mesh=mesh,
        compiler_params=cp,
        out_type=jax.ShapeDtypeStruct((NC, n_pad, d), jnp.float32),
        scratch_types=[
            pltpu.VMEM_SHARED((n_pad, d), jnp.float32),
            pltpu.VMEM((n_chunks, GK), jnp.int32),
            pltpu.VMEM((n_chunks, GK), jnp.int32),
            pltpu.VMEM((n_chunks, GK), jnp.float32),
            pltpu.VMEM((GK, d), jnp.float32),
        ],
    )
    def agg_kernel(x_hbm, src_hbm, dst_hbm, ew_hbm, part_hbm,
                   acc_sh, src_v, dst_v, ew_v, rows_v):
        c = lax.axis_index("c")
        s = lax.axis_index("s")
        gwid = c * NS + s

        @pl.loop(0, GK)
        def _(r):
            for ch in range(d // LANES):
                rows_v[r, pl.ds(ch * LANES, LANES)] = jnp.zeros((LANES,), jnp.float32)

        base = s * rows_per_tile
        for k in range(rows_per_tile // GK):
            pltpu.sync_copy(rows_v, acc_sh.at[pl.ds(base + k * GK, GK)])
        rem = rows_per_tile % GK
        if rem:
            pltpu.sync_copy(rows_v.at[pl.ds(0, rem)],
                            acc_sh.at[pl.ds(base + rows_per_tile - rem, rem)])
        plsc.subcore_barrier()

        pltpu.sync_copy(src_hbm.at[gwid], src_v)
        pltpu.sync_copy(dst_hbm.at[gwid], dst_v)
        pltpu.sync_copy(ew_hbm.at[gwid], ew_v)

        @pl.loop(0, n_chunks)
        def _(i):
            pltpu.sync_copy(x_hbm.at[src_v.at[i]], rows_v)

            @pl.loop(0, GK)
            def _(e):
                wvec = plsc.load_gather(
                    ew_v, [jnp.full((LANES,), i, jnp.int32),
                           jnp.full((LANES,), e, jnp.int32)])
                for ch in range(d // LANES):
                    sl = pl.ds(ch * LANES, LANES)
                    rows_v[e, sl] = rows_v[e, sl] * wvec

            pltpu.sync_copy(rows_v, acc_sh.at[dst_v.at[i]], add=True)

        plsc.subcore_barrier()
        pltpu.sync_copy(acc_sh.at[pl.ds(base, rows_per_tile)],
                        part_hbm.at[c].at[pl.ds(base, rows_per_tile)])

    return agg_kernel(x, src, dst, ew)


def _tc_finish(parts, W, bias, n_out, blk=1000):
    d = parts.shape[2]
    d_out = W.shape[1]

    def body(p_ref, w_ref, b_ref, o_ref):
        agg = p_ref[0] + p_ref[1]
        o_ref[...] = jnp.dot(agg, w_ref[...],
                             preferred_element_type=jnp.float32) + b_ref[...]

    return pl.pallas_call(
        body,
        grid=(n_out // blk,),
        in_specs=[
            pl.BlockSpec((NC, blk, d), lambda i: (0, i, 0)),
            pl.BlockSpec((d, d_out), lambda i: (0, 0)),
            pl.BlockSpec((1, d_out), lambda i: (0, 0)),
        ],
        out_specs=pl.BlockSpec((blk, d_out), lambda i: (i, 0)),
        out_shape=jax.ShapeDtypeStruct((n_out, d_out), jnp.float32),
    )(parts, W, bias.reshape(1, d_out))


def kernel(x, edge_index, edge_weight, W, bias):
    n, d = x.shape
    e = edge_weight.shape[0]
    nw = NC * NS

    e_pad = ((e + nw * GK - 1) // (nw * GK)) * nw * GK
    pad = e_pad - e
    n_chunks = e_pad // (nw * GK)
    dst = jnp.pad(edge_index[0], (0, pad)).reshape(nw, n_chunks, GK)
    src = jnp.pad(edge_index[1], (0, pad)).reshape(nw, n_chunks, GK)
    ew = jnp.pad(edge_weight, (0, pad)).reshape(nw, n_chunks, GK)

    n_pad = ((n + NS * 8 - 1) // (NS * 8)) * NS * 8
    parts = _sc_aggregate(x, src, dst, ew, n_pad, d, n_chunks)
    return _tc_finish(parts, W, bias, n)
